# Initial kernel scaffold; baseline (speedup 1.0000x reference)
import jax
import jax.numpy as jnp
from jax.experimental import pallas as pl


def kernel(feat, topk_values, topk_indices, edge_index, W_neigh, W_self, b_self):
    raise NotImplementedError("write your pallas kernel here")


# speedup vs baseline: 12.7316x; 12.7316x over previous
import functools

import jax
import jax.numpy as jnp
from jax import lax
from jax.experimental import pallas as pl
from jax.experimental.pallas import tpu as pltpu
from jax.experimental.pallas import tpu_sc as plsc

N = 10000
D = 128
K = 32
E = 320000

NPAD = 10240
NW = 32
RD = NPAD // NW
RZ = NPAD // 16
CHUNK = 128
EPT = 10240
NCHUNK = EPT // CHUNK
EPAD = NW * EPT

@functools.cache
def _sc_kernels():
    mesh = plsc.VectorSubcoreMesh(core_axis_name="c", subcore_axis_name="s")
    densify = functools.partial(
        pl.kernel,
        out_type=jax.ShapeDtypeStruct((NPAD * D,), jnp.float32),
        mesh=mesh,
        scratch_types=[
            pltpu.VMEM((RD * D,), jnp.float32),
            pltpu.VMEM((RD, K), jnp.int32),
            pltpu.VMEM((RD, K), jnp.float32),
        ],
        compiler_params=pltpu.CompilerParams(
            needs_layout_passes=False, use_tc_tiling_on_sc=False
        ),
    )(_densify_body)
    spmm = functools.partial(
        pl.kernel,
        out_type=[
            jax.ShapeDtypeStruct((2, NPAD, D), jnp.float32),
            jax.ShapeDtypeStruct((2, NPAD, 16), jnp.float32),
        ],
        mesh=mesh,
        scratch_types=[
            pltpu.VMEM_SHARED((NPAD, D), jnp.float32),
            pltpu.VMEM_SHARED((NPAD, 16), jnp.float32),
            pltpu.VMEM((CHUNK, D), jnp.float32),
            pltpu.VMEM((NCHUNK, CHUNK), jnp.int32),
            pltpu.VMEM((NCHUNK, CHUNK), jnp.int32),
            pltpu.VMEM((CHUNK, 16), jnp.float32),
            pltpu.SemaphoreType.DMA,
        ],
        compiler_params=pltpu.CompilerParams(
            needs_layout_passes=False, use_tc_tiling_on_sc=False
        ),
    )(_spmm_body)
    return densify, spmm


def _densify_body(idx_hbm, val_hbm, dense_hbm, buf, idxv, valv):
    c = lax.axis_index("c")
    s = lax.axis_index("s")
    wid = s * 2 + c
    base = wid * RD

    @pl.loop(0, RD * D // 16)
    def _zero(i):
        buf[pl.ds(i * 16, 16)] = jnp.zeros((16,), jnp.float32)

    pltpu.sync_copy(idx_hbm.at[pl.ds(base, RD)], idxv)
    pltpu.sync_copy(val_hbm.at[pl.ds(base, RD)], valv)

    @pl.loop(0, RD)
    def _scatter(r):
        roff = jnp.full((16,), 0, jnp.int32) + r * D
        for h in range(K // 16):
            iv = idxv[r, pl.ds(h * 16, 16)]
            vv = valv[r, pl.ds(h * 16, 16)]
            plsc.store_scatter(buf, [roff + iv], vv)

    pltpu.sync_copy(buf, dense_hbm.at[pl.ds(base * D, RD * D)])


def _spmm_body(dense_hbm, src_hbm, dst_hbm, part_hbm, cnt_hbm,
               acc, cnt, rows, srcv, dstv, ones, sem):
    c = lax.axis_index("c")
    s = lax.axis_index("s")
    wid = s * 2 + c

    @pl.loop(0, CHUNK)
    def _zr(r):
        for h in range(D // 16):
            rows[r, pl.ds(h * 16, 16)] = jnp.zeros((16,), jnp.float32)

    @pl.loop(0, CHUNK)
    def _zo(r):
        ones[r, :] = jnp.zeros((16,), jnp.float32)

    @pl.loop(0, RZ // CHUNK)
    def _za(i):
        pltpu.sync_copy(rows, acc.at[pl.ds(s * RZ + i * CHUNK, CHUNK)])
        pltpu.sync_copy(ones, cnt.at[pl.ds(s * RZ + i * CHUNK, CHUNK)])

    @pl.loop(0, CHUNK)
    def _so(r):
        ones[r, :] = jnp.zeros((16,), jnp.float32) + 1.0

    plsc.subcore_barrier()

    pltpu.sync_copy(src_hbm.at[wid], srcv)
    pltpu.sync_copy(dst_hbm.at[wid], dstv)

    @pl.loop(0, NCHUNK)
    def _edges(g):
        pltpu.async_copy(dense_hbm.at[srcv.at[g]], rows, sem).wait()
        pltpu.sync_copy(rows, acc.at[dstv.at[g]], add=True)
        pltpu.sync_copy(ones, cnt.at[dstv.at[g]], add=True)

    plsc.subcore_barrier()

    @pl.loop(0, RZ // CHUNK)
    def _wb(i):
        b = s * RZ + i * CHUNK
        pltpu.sync_copy(acc.at[pl.ds(b, CHUNK)], rows)
        pltpu.sync_copy(rows, part_hbm.at[c, pl.ds(b, CHUNK)])
        pltpu.sync_copy(cnt.at[pl.ds(b, CHUNK)], ones)
        pltpu.sync_copy(ones, cnt_hbm.at[c, pl.ds(b, CHUNK)])


BLK = 1280


def _final_body(feat_ref, p_ref, c_ref, wn_ref, ws_ref, b_ref, o_ref):
    cnt = c_ref[0] + c_ref[1]
    deg = jnp.maximum(cnt[:, 0:1], 1.0)
    agg = (p_ref[0] + p_ref[1]) / deg
    hn = lax.dot_general(agg, wn_ref[...], (((1,), (1,)), ((), ())),
                         preferred_element_type=jnp.float32)
    hs = lax.dot_general(feat_ref[...], ws_ref[...], (((1,), (1,)), ((), ())),
                         preferred_element_type=jnp.float32)
    o_ref[...] = hs + b_ref[...] + hn


def _final(featp, part, cntp, w_neigh, w_self, b_self):
    return pl.pallas_call(
        _final_body,
        grid=(NPAD // BLK,),
        in_specs=[
            pl.BlockSpec((BLK, D), lambda i: (i, 0)),
            pl.BlockSpec((2, BLK, D), lambda i: (0, i, 0)),
            pl.BlockSpec((2, BLK, 16), lambda i: (0, i, 0)),
            pl.BlockSpec((D, D), lambda i: (0, 0)),
            pl.BlockSpec((D, D), lambda i: (0, 0)),
            pl.BlockSpec((1, D), lambda i: (0, 0)),
        ],
        out_specs=pl.BlockSpec((BLK, D), lambda i: (i, 0)),
        out_shape=jax.ShapeDtypeStruct((NPAD, D), jnp.float32),
    )(featp, part, cntp, w_neigh, w_self, b_self.reshape(1, D))


def kernel(feat, topk_values, topk_indices, edge_index, W_neigh, W_self, b_self):
    ti = jnp.pad(topk_indices.astype(jnp.int32), ((0, NPAD - N), (0, 0)))
    tv = jnp.pad(topk_values, ((0, NPAD - N), (0, 0)))

    src = edge_index[0].astype(jnp.int32)
    dst = edge_index[1].astype(jnp.int32)
    pad_e = EPAD - E
    pad_src = jnp.arange(pad_e, dtype=jnp.int32) % N
    pad_dst = N + jnp.arange(pad_e, dtype=jnp.int32) % (NPAD - N)
    src_p = jnp.concatenate([src, pad_src]).reshape(NW, NCHUNK, CHUNK)
    dst_p = jnp.concatenate([dst, pad_dst]).reshape(NW, NCHUNK, CHUNK)

    densify, spmm = _sc_kernels()
    dense = densify(ti, tv).reshape(NPAD, D)
    part, cntp = spmm(dense, src_p, dst_p)

    featp = jnp.pad(feat, ((0, NPAD - N), (0, 0)))
    out = _final(featp, part, cntp, W_neigh, W_self, b_self)
    return out[:N]
